# baseline (speedup 1.0000x reference)
import functools

import jax
import jax.numpy as jnp
from jax import lax
from jax.experimental import pallas as pl
from jax.experimental.pallas import tpu as pltpu
from jax.experimental.pallas import tpu_sc as plsc

C = 21
S = 32
NBINS = S * S
L = 16
NC, NS = 2, 16
NW = NC * NS
N = 4 * 512 * 512
PER_W = N // NW
NR = NBINS // L
CHUNK = 8192
NCH = PER_W // CHUNK
CV = CHUNK // L

_mesh = plsc.VectorSubcoreMesh(core_axis_name="c", subcore_axis_name="s")


@functools.partial(
    pl.kernel,
    out_type=jax.ShapeDtypeStruct((NW, NR, L), jnp.float32),
    mesh=_mesh,
    compiler_params=pltpu.CompilerParams(needs_layout_passes=False),
    scratch_types=[
        pltpu.VMEM((2, CHUNK), jnp.int32),
        pltpu.VMEM((2, CHUNK), jnp.int32),
        pltpu.VMEM((L * NBINS,), jnp.float32),
        pltpu.VMEM((NR, L), jnp.float32),
        pltpu.SemaphoreType.DMA,
        pltpu.SemaphoreType.DMA,
    ],
)
def _cm_sc_kernel(yt_hbm, yp_hbm, out_hbm, yt_v, yp_v, acc, part, sem0, sem1):
    cid = lax.axis_index("c")
    sid = lax.axis_index("s")
    wid = sid * NC + cid
    base = wid * PER_W
    sems = (sem0, sem1)

    def stage(c, slot):
        off = base + c * CHUNK
        pltpu.make_async_copy(
            yt_hbm.at[pl.ds(off, CHUNK)], yt_v.at[slot], sems[slot]).start()
        pltpu.make_async_copy(
            yp_hbm.at[pl.ds(off, CHUNK)], yp_v.at[slot], sems[slot]).start()

    def drain(c, slot):
        off = base + c * CHUNK
        pltpu.make_async_copy(
            yt_hbm.at[pl.ds(off, CHUNK)], yt_v.at[slot], sems[slot]).wait()
        pltpu.make_async_copy(
            yp_hbm.at[pl.ds(off, CHUNK)], yp_v.at[slot], sems[slot]).wait()

    stage(0, 0)

    zeros = jnp.zeros((L,), jnp.float32)
    iota = lax.iota(jnp.int32, L)

    def zero_acc(j, carry):
        acc[pl.ds(j * L, L)] = zeros
        return carry

    lax.fori_loop(0, NBINS, zero_acc, 0)

    lane_off = iota * NBINS
    ones = jnp.ones((L,), jnp.float32)

    for c in range(NCH):
        slot = c % 2
        if c + 1 < NCH:
            stage(c + 1, 1 - slot)
        drain(c, slot)

        def body(i, carry):
            yt = yt_v[slot, pl.ds(i * L, L)]
            yp = yp_v[slot, pl.ds(i * L, L)]
            b = yt * S + yp + lane_off
            plsc.addupdate_scatter(acc, [b], ones)
            return carry

        lax.fori_loop(0, CV, body, 0)

    def reduce_lanes(j, carry):
        s = acc[pl.ds(j * L, L)]
        for l in range(1, L):
            s = s + acc[pl.ds(l * NBINS + j * L, L)]
        part[j] = s
        return carry

    lax.fori_loop(0, NR, reduce_lanes, 0)

    pltpu.sync_copy(part, out_hbm.at[wid])


def _combine_tc_kernel(parts_ref, cm_ref, out_ref):
    total = jnp.sum(parts_ref[...], axis=0)
    row = lax.broadcasted_iota(jnp.int32, (S, S), 0)
    total = jnp.where(row == 0, 0.0, total)
    out_ref[...] = total[:C, :C] + cm_ref[...]


_combine_tc = pl.pallas_call(
    _combine_tc_kernel,
    out_shape=jax.ShapeDtypeStruct((C, C), jnp.float32),
)


def kernel(y_true, y_pred, cm):
    yt = y_true.reshape(-1)
    yp = y_pred.reshape(-1)
    parts = _cm_sc_kernel(yt, yp)
    return _combine_tc(parts.reshape(NW, S, S), cm)


# speedup vs baseline: 1.0967x; 1.0967x over previous
import functools

import jax
import jax.numpy as jnp
from jax import lax
from jax.experimental import pallas as pl
from jax.experimental.pallas import tpu as pltpu
from jax.experimental.pallas import tpu_sc as plsc

C = 21
S = 32
NBINS = S * S
L = 16
NC, NS = 2, 16
NW = NC * NS
N = 4 * 512 * 512
PER_W = N // NW
NR = NBINS // L
CHUNK = 8192
NCH = PER_W // CHUNK
CV = CHUNK // L

_mesh = plsc.VectorSubcoreMesh(core_axis_name="c", subcore_axis_name="s")


@functools.partial(
    pl.kernel,
    out_type=jax.ShapeDtypeStruct((NW, NR, L), jnp.float32),
    mesh=_mesh,
    compiler_params=pltpu.CompilerParams(needs_layout_passes=False),
    scratch_types=[
        pltpu.VMEM((2, CHUNK), jnp.int32),
        pltpu.VMEM((2, CHUNK), jnp.int32),
        pltpu.VMEM((L * NBINS,), jnp.float32),
        pltpu.VMEM((NR, L), jnp.float32),
        pltpu.SemaphoreType.DMA,
        pltpu.SemaphoreType.DMA,
    ],
)
def _cm_sc_kernel(yt_hbm, yp_hbm, out_hbm, yt_v, yp_v, acc, part, sem0, sem1):
    cid = lax.axis_index("c")
    sid = lax.axis_index("s")
    wid = sid * NC + cid
    base = wid * PER_W
    sems = (sem0, sem1)

    def stage(c, slot):
        off = base + c * CHUNK
        pltpu.make_async_copy(
            yt_hbm.at[pl.ds(off, CHUNK)], yt_v.at[slot], sems[slot]).start()
        pltpu.make_async_copy(
            yp_hbm.at[pl.ds(off, CHUNK)], yp_v.at[slot], sems[slot]).start()

    def drain(c, slot):
        off = base + c * CHUNK
        pltpu.make_async_copy(
            yt_hbm.at[pl.ds(off, CHUNK)], yt_v.at[slot], sems[slot]).wait()
        pltpu.make_async_copy(
            yp_hbm.at[pl.ds(off, CHUNK)], yp_v.at[slot], sems[slot]).wait()

    stage(0, 0)

    zeros = jnp.zeros((L,), jnp.float32)
    iota = lax.iota(jnp.int32, L)

    ZU = 8

    def zero_acc(j, carry):
        for k in range(ZU):
            acc[pl.ds((j * ZU + k) * L, L)] = zeros
        return carry

    lax.fori_loop(0, NBINS // ZU, zero_acc, 0)

    lane_off = iota * NBINS
    ones = jnp.ones((L,), jnp.float32)

    for c in range(NCH):
        slot = c % 2
        if c + 1 < NCH:
            stage(c + 1, 1 - slot)
        drain(c, slot)

        U = 8

        def body(i, carry):
            for k in range(U):
                yt = yt_v[slot, pl.ds((i * U + k) * L, L)]
                yp = yp_v[slot, pl.ds((i * U + k) * L, L)]
                b = yt * S + yp + lane_off
                plsc.addupdate_scatter(acc, [b], ones)
            return carry

        lax.fori_loop(0, CV // U, body, 0)

    def reduce_lanes(j, carry):
        s = acc[pl.ds(j * L, L)]
        for l in range(1, L):
            s = s + acc[pl.ds(l * NBINS + j * L, L)]
        part[j] = s
        return carry

    lax.fori_loop(0, NR, reduce_lanes, 0)

    pltpu.sync_copy(part, out_hbm.at[wid])


def _combine_tc_kernel(parts_ref, cm_ref, out_ref):
    total = jnp.sum(parts_ref[...], axis=0)
    row = lax.broadcasted_iota(jnp.int32, (S, S), 0)
    total = jnp.where(row == 0, 0.0, total)
    out_ref[...] = total[:C, :C] + cm_ref[...]


_combine_tc = pl.pallas_call(
    _combine_tc_kernel,
    out_shape=jax.ShapeDtypeStruct((C, C), jnp.float32),
)


def kernel(y_true, y_pred, cm):
    yt = y_true.reshape(-1)
    yp = y_pred.reshape(-1)
    parts = _cm_sc_kernel(yt, yp)
    return _combine_tc(parts.reshape(NW, S, S), cm)


# speedup vs baseline: 1.4462x; 1.3187x over previous
import functools

import jax
import jax.numpy as jnp
from jax import lax
from jax.experimental import pallas as pl
from jax.experimental.pallas import tpu as pltpu
from jax.experimental.pallas import tpu_sc as plsc

C = 21
S = 32
NBINS = S * S
L = 16
NC, NS = 2, 16
NW = NC * NS
N = 4 * 512 * 512
PER_W = N // NW
NR = NBINS // L
CHUNK = 8192
NCH = PER_W // CHUNK
CV = CHUNK // L

_mesh = plsc.VectorSubcoreMesh(core_axis_name="c", subcore_axis_name="s")


@functools.partial(
    pl.kernel,
    out_type=jax.ShapeDtypeStruct((NW, NR, L), jnp.float32),
    mesh=_mesh,
    compiler_params=pltpu.CompilerParams(needs_layout_passes=False),
    scratch_types=[
        pltpu.VMEM((2, CHUNK), jnp.int32),
        pltpu.VMEM((2, CHUNK), jnp.int32),
        pltpu.VMEM((L * NBINS,), jnp.float32),
        pltpu.VMEM((NR, L), jnp.float32),
        pltpu.SemaphoreType.DMA,
        pltpu.SemaphoreType.DMA,
    ],
)
def _cm_sc_kernel(yt_hbm, yp_hbm, out_hbm, yt_v, yp_v, acc, part, sem0, sem1):
    cid = lax.axis_index("c")
    sid = lax.axis_index("s")
    wid = sid * NC + cid
    base = wid * PER_W
    sems = (sem0, sem1)

    def stage(c, slot):
        off = base + c * CHUNK
        pltpu.make_async_copy(
            yt_hbm.at[pl.ds(off, CHUNK)], yt_v.at[slot], sems[slot]).start()
        pltpu.make_async_copy(
            yp_hbm.at[pl.ds(off, CHUNK)], yp_v.at[slot], sems[slot]).start()

    def drain(c, slot):
        off = base + c * CHUNK
        pltpu.make_async_copy(
            yt_hbm.at[pl.ds(off, CHUNK)], yt_v.at[slot], sems[slot]).wait()
        pltpu.make_async_copy(
            yp_hbm.at[pl.ds(off, CHUNK)], yp_v.at[slot], sems[slot]).wait()

    stage(0, 0)

    zeros = jnp.zeros((L,), jnp.float32)
    iota = lax.iota(jnp.int32, L)

    @plsc.parallel_loop(0, NBINS, unroll=8)
    def _(j):
        acc[pl.ds(j * L, L)] = zeros

    lane_off = iota * NBINS
    ones = jnp.ones((L,), jnp.float32)

    for c in range(NCH):
        slot = c % 2
        if c + 1 < NCH:
            stage(c + 1, 1 - slot)
        drain(c, slot)

        @plsc.parallel_loop(0, CV, unroll=8)
        def _(i):
            yt = yt_v[slot, pl.ds(i * L, L)]
            yp = yp_v[slot, pl.ds(i * L, L)]
            b = yt * S + yp + lane_off
            plsc.addupdate_scatter(acc, [b], ones)

    @plsc.parallel_loop(0, NR, unroll=2)
    def _(j):
        s = acc[pl.ds(j * L, L)]
        for l in range(1, L):
            s = s + acc[pl.ds(l * NBINS + j * L, L)]
        part[j] = s

    pltpu.sync_copy(part, out_hbm.at[wid])


def _combine_tc_kernel(parts_ref, cm_ref, out_ref):
    total = jnp.sum(parts_ref[...], axis=0)
    row = lax.broadcasted_iota(jnp.int32, (S, S), 0)
    total = jnp.where(row == 0, 0.0, total)
    out_ref[...] = total[:C, :C] + cm_ref[...]


_combine_tc = pl.pallas_call(
    _combine_tc_kernel,
    out_shape=jax.ShapeDtypeStruct((C, C), jnp.float32),
)


def kernel(y_true, y_pred, cm):
    yt = y_true.reshape(-1)
    yp = y_pred.reshape(-1)
    parts = _cm_sc_kernel(yt, yp)
    return _combine_tc(parts.reshape(NW, S, S), cm)


# speedup vs baseline: 1.4497x; 1.0024x over previous
import functools

import jax
import jax.numpy as jnp
from jax import lax
from jax.experimental import pallas as pl
from jax.experimental.pallas import tpu as pltpu
from jax.experimental.pallas import tpu_sc as plsc

C = 21
S = 32
NBINS = S * S
L = 16
NC, NS = 2, 16
NW = NC * NS
N = 4 * 512 * 512
PER_W = N // NW
NR = NBINS // L
CHUNK = 8192
NCH = PER_W // CHUNK
CV = CHUNK // L

_mesh = plsc.VectorSubcoreMesh(core_axis_name="c", subcore_axis_name="s")


@functools.partial(
    pl.kernel,
    out_type=jax.ShapeDtypeStruct((NW, NR, L), jnp.float32),
    mesh=_mesh,
    compiler_params=pltpu.CompilerParams(
        needs_layout_passes=False, use_tc_tiling_on_sc=True),
    scratch_types=[
        pltpu.VMEM((2, CHUNK), jnp.int32),
        pltpu.VMEM((2, CHUNK), jnp.int32),
        pltpu.VMEM((L * NBINS,), jnp.float32),
        pltpu.VMEM((NR, L), jnp.float32),
        pltpu.SemaphoreType.DMA,
        pltpu.SemaphoreType.DMA,
    ],
)
def _cm_sc_kernel(yt_hbm, yp_hbm, out_hbm, yt_v, yp_v, acc, part, sem0, sem1):
    cid = lax.axis_index("c")
    sid = lax.axis_index("s")
    wid = sid * NC + cid
    base = wid * PER_W
    sems = (sem0, sem1)

    def stage(c, slot):
        off = base + c * CHUNK
        pltpu.make_async_copy(
            yt_hbm.at[pl.ds(off, CHUNK)], yt_v.at[slot], sems[slot]).start()
        pltpu.make_async_copy(
            yp_hbm.at[pl.ds(off, CHUNK)], yp_v.at[slot], sems[slot]).start()

    def drain(c, slot):
        off = base + c * CHUNK
        pltpu.make_async_copy(
            yt_hbm.at[pl.ds(off, CHUNK)], yt_v.at[slot], sems[slot]).wait()
        pltpu.make_async_copy(
            yp_hbm.at[pl.ds(off, CHUNK)], yp_v.at[slot], sems[slot]).wait()

    stage(0, 0)

    zeros = jnp.zeros((L,), jnp.float32)
    iota = lax.iota(jnp.int32, L)

    @plsc.parallel_loop(0, NBINS, unroll=8)
    def _(j):
        acc[pl.ds(j * L, L)] = zeros

    lane_off = iota * NBINS
    ones = jnp.ones((L,), jnp.float32)

    for c in range(NCH):
        slot = c % 2
        if c + 1 < NCH:
            stage(c + 1, 1 - slot)
        drain(c, slot)

        @plsc.parallel_loop(0, CV, unroll=8)
        def _(i):
            yt = yt_v[slot, pl.ds(i * L, L)]
            yp = yp_v[slot, pl.ds(i * L, L)]
            b = yt * S + yp + lane_off
            plsc.addupdate_scatter(acc, [b], ones)

    @plsc.parallel_loop(0, NR, unroll=2)
    def _(j):
        s = acc[pl.ds(j * L, L)]
        for l in range(1, L):
            s = s + acc[pl.ds(l * NBINS + j * L, L)]
        part[j] = s

    pltpu.sync_copy(part, out_hbm.at[wid])


def _combine_tc_kernel(parts_ref, cm_ref, out_ref):
    total = jnp.sum(parts_ref[...], axis=0)
    row = lax.broadcasted_iota(jnp.int32, (S, S), 0)
    total = jnp.where(row == 0, 0.0, total)
    out_ref[...] = total[:C, :C] + cm_ref[...]


_combine_tc = pl.pallas_call(
    _combine_tc_kernel,
    out_shape=jax.ShapeDtypeStruct((C, C), jnp.float32),
)


def kernel(y_true, y_pred, cm):
    yt = y_true.reshape(-1)
    yp = y_pred.reshape(-1)
    parts = _cm_sc_kernel(yt, yp)
    return _combine_tc(parts.reshape(NW, S, S), cm)


# speedup vs baseline: 1.7400x; 1.2002x over previous
import functools

import jax
import jax.numpy as jnp
from jax import lax
from jax.experimental import pallas as pl
from jax.experimental.pallas import tpu as pltpu
from jax.experimental.pallas import tpu_sc as plsc

C = 21
S = 32
NBINS = S * S
L = 16
NC, NS = 2, 16
NW = NC * NS
N = 4 * 512 * 512
W = 512
ROWS = N // W
PER_W = N // NW
RPW = PER_W // W
NR = NBINS // L
CHUNK = 8192
CR = CHUNK // W
NCH = PER_W // CHUNK
CV = CHUNK // L

_mesh = plsc.VectorSubcoreMesh(core_axis_name="c", subcore_axis_name="s")


@functools.partial(
    pl.kernel,
    out_type=jax.ShapeDtypeStruct((NW, NR, L), jnp.float32),
    mesh=_mesh,
    compiler_params=pltpu.CompilerParams(
        needs_layout_passes=False, use_tc_tiling_on_sc=True),
    scratch_types=[
        pltpu.VMEM((2, CR, W), jnp.int32),
        pltpu.VMEM((2, CR, W), jnp.int32),
        pltpu.VMEM((L * NBINS,), jnp.float32),
        pltpu.VMEM((NR, L), jnp.float32),
        pltpu.SemaphoreType.DMA,
        pltpu.SemaphoreType.DMA,
    ],
)
def _cm_sc_kernel(yt_hbm, yp_hbm, out_hbm, yt_v, yp_v, acc, part, sem0, sem1):
    cid = lax.axis_index("c")
    sid = lax.axis_index("s")
    wid = sid * NC + cid
    base = wid * RPW
    sems = (sem0, sem1)

    def stage(c, slot):
        off = base + c * CR
        pltpu.make_async_copy(
            yt_hbm.at[pl.ds(off, CR)], yt_v.at[slot], sems[slot]).start()
        pltpu.make_async_copy(
            yp_hbm.at[pl.ds(off, CR)], yp_v.at[slot], sems[slot]).start()

    def drain(c, slot):
        off = base + c * CR
        pltpu.make_async_copy(
            yt_hbm.at[pl.ds(off, CR)], yt_v.at[slot], sems[slot]).wait()
        pltpu.make_async_copy(
            yp_hbm.at[pl.ds(off, CR)], yp_v.at[slot], sems[slot]).wait()

    stage(0, 0)

    zeros = jnp.zeros((L,), jnp.float32)
    iota = lax.iota(jnp.int32, L)

    @plsc.parallel_loop(0, NBINS, unroll=8)
    def _(j):
        acc[pl.ds(j * L, L)] = zeros

    lane_off = iota * NBINS
    ones = jnp.ones((L,), jnp.float32)

    for c in range(NCH):
        slot = c % 2
        if c + 1 < NCH:
            stage(c + 1, 1 - slot)
        drain(c, slot)

        @plsc.parallel_loop(0, CV, unroll=8)
        def _(i):
            r = lax.shift_right_logical(i, 5)
            cc = lax.shift_left(lax.bitwise_and(i, 31), 4)
            yt = yt_v[slot, r, pl.ds(cc, L)]
            yp = yp_v[slot, r, pl.ds(cc, L)]
            b = yt * S + yp + lane_off
            plsc.addupdate_scatter(acc, [b], ones)

    @plsc.parallel_loop(0, NR, unroll=2)
    def _(j):
        s = acc[pl.ds(j * L, L)]
        for l in range(1, L):
            s = s + acc[pl.ds(l * NBINS + j * L, L)]
        part[j] = s

    pltpu.sync_copy(part, out_hbm.at[wid])


def _combine_tc_kernel(parts_ref, cm_ref, out_ref):
    total = jnp.sum(parts_ref[...], axis=0)
    row = lax.broadcasted_iota(jnp.int32, (S, S), 0)
    total = jnp.where(row == 0, 0.0, total)
    out_ref[...] = total[:C, :C] + cm_ref[...]


_combine_tc = pl.pallas_call(
    _combine_tc_kernel,
    out_shape=jax.ShapeDtypeStruct((C, C), jnp.float32),
)


def kernel(y_true, y_pred, cm):
    yt = y_true.reshape(ROWS, W)
    yp = y_pred.reshape(ROWS, W)
    parts = _cm_sc_kernel(yt, yp)
    return _combine_tc(parts.reshape(NW, S, S), cm)
